# Initial kernel scaffold; baseline (speedup 1.0000x reference)
import jax
import jax.numpy as jnp
from jax.experimental import pallas as pl


def kernel(x, wts):
    raise NotImplementedError("write your pallas kernel here")


# speedup vs baseline: 5.1826x; 5.1826x over previous
import functools

import jax
import jax.numpy as jnp
from jax import lax
from jax.experimental import pallas as pl
from jax.experimental.pallas import tpu as pltpu
from jax.experimental.pallas import tpu_sc as plsc

INPUT_DIM = 100000
EMBED_DIM = 128
BATCH = 4096
SEQ = 200

NUM_CORES = 2
NUM_SUBCORES = 16
NW = NUM_CORES * NUM_SUBCORES

TOTAL = BATCH * SEQ
B_PER_W = TOTAL // NW
CHUNK = 128
STEPS = B_PER_W // CHUNK


def _embed_kernel(idx_hbm, tbl_hbm, out_hbm, idx_v, rows_v, sem_i, sem_g):
    wid = lax.axis_index("s") * NUM_CORES + lax.axis_index("c")
    base = wid * B_PER_W

    def step(i, carry):
        off = base + i * CHUNK
        pltpu.sync_copy(idx_hbm.at[pl.ds(off, CHUNK)], idx_v)
        pltpu.async_copy(tbl_hbm.at[idx_v], rows_v, sem_g).wait()
        pltpu.sync_copy(rows_v, out_hbm.at[pl.ds(off, CHUNK)])
        return carry

    lax.fori_loop(0, STEPS, step, 0)


@jax.jit
def _embed(x_flat, wts):
    run = pl.kernel(
        _embed_kernel,
        out_type=jax.ShapeDtypeStruct((TOTAL, EMBED_DIM), jnp.float32),
        mesh=plsc.VectorSubcoreMesh(core_axis_name="c", subcore_axis_name="s"),
        scratch_types=[
            pltpu.VMEM((CHUNK,), jnp.int32),
            pltpu.VMEM((CHUNK, EMBED_DIM), jnp.float32),
            pltpu.SemaphoreType.DMA,
            pltpu.SemaphoreType.DMA,
        ],
    )
    return run(x_flat, wts)


def kernel(x, wts):
    out = _embed(x.reshape(-1), wts)
    return out.reshape(BATCH, SEQ, EMBED_DIM)


# speedup vs baseline: 9.2019x; 1.7756x over previous
import functools

import jax
import jax.numpy as jnp
from jax import lax
from jax.experimental import pallas as pl
from jax.experimental.pallas import tpu as pltpu
from jax.experimental.pallas import tpu_sc as plsc

INPUT_DIM = 100000
EMBED_DIM = 128
BATCH = 4096
SEQ = 200

NUM_CORES = 2
NUM_SUBCORES = 16
NW = NUM_CORES * NUM_SUBCORES

TOTAL = BATCH * SEQ
B_PER_W = TOTAL // NW
CHUNK = 128
STEPS = B_PER_W // CHUNK
NBUF = 4


def _embed_kernel(idx_hbm, tbl_hbm, out_hbm, idx_all, rows, sem_g, sem_o):
    wid = lax.axis_index("s") * NUM_CORES + lax.axis_index("c")
    base = wid * B_PER_W

    pltpu.sync_copy(idx_hbm.at[wid], idx_all)

    def start_gather(i, b):
        pltpu.async_copy(tbl_hbm.at[idx_all.at[i]], rows[b], sem_g[b])

    def wait_gather(i, b):
        pltpu.make_async_copy(tbl_hbm.at[idx_all.at[i]], rows[b],
                              sem_g[b]).wait()

    for b in range(NBUF):
        start_gather(b, b)

    def it_body(it, carry):
        for b in range(NBUF):
            i = it * NBUF + b
            wait_gather(i, b)
            pltpu.async_copy(
                rows[b], out_hbm.at[pl.ds(base + i * CHUNK, CHUNK)],
                sem_o[b]).wait()
            start_gather(i + NBUF, b)
        return carry

    lax.fori_loop(0, STEPS // NBUF - 1, it_body, 0)

    for b in range(NBUF):
        i = STEPS - NBUF + b
        wait_gather(i, b)
        pltpu.sync_copy(rows[b], out_hbm.at[pl.ds(base + i * CHUNK, CHUNK)])


@jax.jit
def _embed(x_blk, wts):
    run = pl.kernel(
        _embed_kernel,
        out_type=jax.ShapeDtypeStruct((TOTAL, EMBED_DIM), jnp.float32),
        mesh=plsc.VectorSubcoreMesh(core_axis_name="c", subcore_axis_name="s"),
        scratch_types=[
            pltpu.VMEM((STEPS, CHUNK), jnp.int32),
            [pltpu.VMEM((CHUNK, EMBED_DIM), jnp.float32)] * NBUF,
            [pltpu.SemaphoreType.DMA] * NBUF,
            [pltpu.SemaphoreType.DMA] * NBUF,
        ],
    )
    return run(x_blk, wts)


def kernel(x, wts):
    out = _embed(x.reshape(NW, STEPS, CHUNK), wts)
    return out.reshape(BATCH, SEQ, EMBED_DIM)


# speedup vs baseline: 9.2248x; 1.0025x over previous
import functools

import jax
import jax.numpy as jnp
from jax import lax
from jax.experimental import pallas as pl
from jax.experimental.pallas import tpu as pltpu
from jax.experimental.pallas import tpu_sc as plsc

INPUT_DIM = 100000
EMBED_DIM = 128
BATCH = 4096
SEQ = 200

NUM_CORES = 2
NUM_SUBCORES = 16
NW = NUM_CORES * NUM_SUBCORES

TOTAL = BATCH * SEQ
B_PER_W = TOTAL // NW
CHUNK = 128
STEPS = B_PER_W // CHUNK
NBUF = 5
GAHEAD = 3
OLAG = 2


def _embed_kernel(idx_hbm, tbl_hbm, out_hbm, idx_all, rows, sem_g, sem_o):
    wid = lax.axis_index("s") * NUM_CORES + lax.axis_index("c")
    base = wid * B_PER_W

    pltpu.sync_copy(idx_hbm.at[wid], idx_all)

    def start_gather(i, b):
        pltpu.async_copy(tbl_hbm.at[idx_all.at[i]], rows[b], sem_g[b])

    def wait_gather(i, b):
        pltpu.make_async_copy(tbl_hbm.at[idx_all.at[i]], rows[b],
                              sem_g[b]).wait()

    def start_out(i, b):
        pltpu.async_copy(rows[b], out_hbm.at[pl.ds(base + i * CHUNK, CHUNK)],
                         sem_o[b])

    def wait_out(i, b):
        pltpu.make_async_copy(rows[b],
                              out_hbm.at[pl.ds(base + i * CHUNK, CHUNK)],
                              sem_o[b]).wait()

    def slot(i, b, do_wait_out, do_start_gather):
        wait_gather(i, b % NBUF)
        start_out(i, b % NBUF)
        if do_wait_out:
            wait_out(i - OLAG, (b - OLAG) % NBUF)
        if do_start_gather:
            start_gather(i + GAHEAD, (b + GAHEAD) % NBUF)

    for b in range(GAHEAD):
        start_gather(b, b)

    for i in range(NBUF):
        slot(i, i, i >= OLAG, True)

    def it_body(it, carry):
        for b in range(NBUF):
            slot(it * NBUF + b, b, True, True)
        return carry

    lax.fori_loop(1, STEPS // NBUF - 1, it_body, 0)

    for i in range(STEPS - NBUF, STEPS):
        slot(i, i, True, i + GAHEAD < STEPS)
    for i in range(STEPS - OLAG, STEPS):
        wait_out(i, i % NBUF)


@jax.jit
def _embed(x_blk, wts):
    run = pl.kernel(
        _embed_kernel,
        out_type=jax.ShapeDtypeStruct((TOTAL, EMBED_DIM), jnp.float32),
        mesh=plsc.VectorSubcoreMesh(core_axis_name="c", subcore_axis_name="s"),
        scratch_types=[
            pltpu.VMEM((STEPS, CHUNK), jnp.int32),
            [pltpu.VMEM((CHUNK, EMBED_DIM), jnp.float32)] * NBUF,
            [pltpu.SemaphoreType.DMA] * NBUF,
            [pltpu.SemaphoreType.DMA] * NBUF,
        ],
    )
    return run(x_blk, wts)


def kernel(x, wts):
    out = _embed(x.reshape(NW, STEPS, CHUNK), wts)
    return out.reshape(BATCH, SEQ, EMBED_DIM)
